# baseline (speedup 1.0000x reference)
import functools

import jax
import jax.numpy as jnp
from jax import lax
from jax.experimental import pallas as pl
from jax.experimental.pallas import tpu as pltpu
from jax.experimental.pallas import tpu_sc as plsc

N_CLS = 100
N_CTX = 16
CTX_DIM = 512
SEQ_LEN = 77
SUFFIX_LEN = SEQ_LEN - 1 - N_CTX
BATCH = 1024

NC, NS, L = 2, 16, 16
NW = NC * NS
BPW = BATCH // NW

TOK_PAD = 80

CTX_Q = 4
SFX_Q = 6
CTX_P = N_CTX // CTX_Q
SFX_P = SUFFIX_LEN // SFX_Q


def _sc_gather_body(label_hbm, pref_hbm, ctx_hbm, sfx_hbm, tok_hbm,
                    out_hbm, tokout_hbm,
                    idx_v, sidx_v, pbuf, cbuf, sbuf, tokbuf, sem):
  wid = lax.axis_index("s") * NC + lax.axis_index("c")
  base = wid * BPW

  pltpu.sync_copy(label_hbm.at[pl.ds(base, BPW)], idx_v)

  for g in range(BPW // L):
    b0 = base + g * L
    idx16 = idx_v.at[pl.ds(g * L, L)]

    pltpu.async_copy(tok_hbm.at[idx16], tokbuf, sem).wait()
    pltpu.sync_copy(tokbuf, tokout_hbm.at[pl.ds(b0, L)])

    pltpu.async_copy(pref_hbm.at[idx16], pbuf, sem).wait()
    pltpu.sync_copy(pbuf, out_hbm.at[pl.ds(b0, L), pl.ds(0, 1)])

    lab = idx_v[pl.ds(g * L, L)]

    for q in range(CTX_Q):
      sidx_v[...] = lab * CTX_Q + q
      pltpu.async_copy(ctx_hbm.at[sidx_v], cbuf, sem).wait()
      pltpu.sync_copy(cbuf, out_hbm.at[pl.ds(b0, L), pl.ds(1 + q * CTX_P, CTX_P)])

    for q in range(SFX_Q):
      sidx_v[...] = lab * SFX_Q + q
      pltpu.async_copy(sfx_hbm.at[sidx_v], sbuf, sem).wait()
      pltpu.sync_copy(sbuf, out_hbm.at[pl.ds(b0, L), pl.ds(1 + N_CTX + q * SFX_P, SFX_P)])


@jax.jit
def _prompt_gather(label, ctx, token_prefix, token_suffix, tokenized_prompts):
  ctx_r = ctx.reshape(N_CLS * CTX_Q, CTX_P, CTX_DIM)
  sfx_r = token_suffix.reshape(N_CLS * SFX_Q, SFX_P, CTX_DIM)
  tok_r = jnp.pad(tokenized_prompts, ((0, 0), (0, TOK_PAD - SEQ_LEN)))

  mesh = plsc.VectorSubcoreMesh(core_axis_name="c", subcore_axis_name="s")
  run = pl.kernel(
      _sc_gather_body,
      out_type=(
          jax.ShapeDtypeStruct((BATCH, SEQ_LEN, CTX_DIM), jnp.float32),
          jax.ShapeDtypeStruct((BATCH, TOK_PAD), jnp.int32),
      ),
      mesh=mesh,
      scratch_types=[
          pltpu.VMEM((BPW,), jnp.int32),
          pltpu.VMEM((L,), jnp.int32),
          pltpu.VMEM((L, 1, CTX_DIM), jnp.float32),
          pltpu.VMEM((L, CTX_P, CTX_DIM), jnp.float32),
          pltpu.VMEM((L, SFX_P, CTX_DIM), jnp.float32),
          pltpu.VMEM((L, TOK_PAD), jnp.int32),
          pltpu.SemaphoreType.DMA,
      ],
      compiler_params=pltpu.CompilerParams(use_tc_tiling_on_sc=False),
  )
  prompts, tok_padded = run(label, token_prefix, ctx_r, sfx_r, tok_r)
  return prompts, tok_padded[:, :SEQ_LEN]


def kernel(label, ctx, token_prefix, token_suffix, tokenized_prompts):
  return _prompt_gather(label, ctx, token_prefix, token_suffix,
                        tokenized_prompts)


# speedup vs baseline: 1.0427x; 1.0427x over previous
import jax
import jax.numpy as jnp
from jax import lax
from jax.experimental import pallas as pl
from jax.experimental.pallas import tpu as pltpu
from jax.experimental.pallas import tpu_sc as plsc

N_CLS = 100
N_CTX = 16
CTX_DIM = 512
SEQ_LEN = 77
SUFFIX_LEN = SEQ_LEN - 1 - N_CTX
BATCH = 1024

NC, NS, L = 2, 16, 16
NW = NC * NS
BPW = BATCH // NW
GROUPS = BPW // L

TOK_PAD = 80

P = 4
CTX_PIECES = N_CTX // P
SFX_PIECES = SUFFIX_LEN // P
PIECES = CTX_PIECES + SFX_PIECES
NBUF = 3


def _sc_gather_body(label_hbm, pref_hbm, ctx_hbm, sfx_hbm, tok_hbm,
                    out_hbm, tokout_hbm,
                    idx_v, sidx_v, pbuf, tokbuf, buf0, buf1, buf2,
                    gsems, ssems, tg_sem, ts_sem, pg_sem, ps_sem):
  wid = lax.axis_index("s") * NC + lax.axis_index("c")
  base = wid * BPW
  bufs = (buf0, buf1, buf2)

  pltpu.sync_copy(label_hbm.at[pl.ds(base, BPW)], idx_v)

  gathers = {}
  scatters = {}
  tokc = prefc = None

  def piece_plan(p):
    if p < CTX_PIECES:
      return ctx_hbm, CTX_PIECES, p, 1 + p * P
    s = p - CTX_PIECES
    return sfx_hbm, SFX_PIECES, s, 1 + N_CTX + s * P

  for g in range(GROUPS):
    b0 = base + g * L
    idx16 = idx_v.at[pl.ds(g * L, L)]

    if g > 0:
      tokc.wait()
      prefc.wait()
    tokg = pltpu.make_async_copy(tok_hbm.at[idx16], tokbuf, tg_sem)
    tokg.start()
    prefg = pltpu.make_async_copy(pref_hbm.at[idx16], pbuf, pg_sem)
    prefg.start()

    lab = idx_v[pl.ds(g * L, L)]

    for p in range(PIECES):
      pc = g * PIECES + p
      r = pc % NBUF
      tbl, mult, off, col = piece_plan(p)
      if pc >= NBUF:
        scatters.pop(pc - NBUF).wait()
      sidx_v[r] = lab * mult + off
      h = pltpu.make_async_copy(tbl.at[sidx_v.at[r]], bufs[r], gsems[r])
      h.start()
      gathers[pc] = h
      if pc >= 1:
        q = (pc - 1) % NBUF
        gathers.pop(pc - 1).wait()
        _, _, _, pcol = piece_plan((pc - 1) % PIECES)
        pb0 = base + ((pc - 1) // PIECES) * L
        sh = pltpu.make_async_copy(
            bufs[q], out_hbm.at[pl.ds(pb0, L), pl.ds(pcol, P)], ssems[q])
        sh.start()
        scatters[pc - 1] = sh

    tokg.wait()
    tokc = pltpu.make_async_copy(tokbuf, tokout_hbm.at[pl.ds(b0, L)], ts_sem)
    tokc.start()
    prefg.wait()
    prefc = pltpu.make_async_copy(
        pbuf, out_hbm.at[pl.ds(b0, L), pl.ds(0, 1)], ps_sem)
    prefc.start()

  last = GROUPS * PIECES - 1
  gathers.pop(last).wait()
  _, _, _, pcol = piece_plan(last % PIECES)
  pb0 = base + (last // PIECES) * L
  sh = pltpu.make_async_copy(
      bufs[last % NBUF], out_hbm.at[pl.ds(pb0, L), pl.ds(pcol, P)],
      ssems[last % NBUF])
  sh.start()
  scatters[last] = sh
  for pc in sorted(scatters):
    scatters[pc].wait()
  tokc.wait()
  prefc.wait()


@jax.jit
def _prompt_gather(label, ctx, token_prefix, token_suffix, tokenized_prompts):
  ctx_r = ctx.reshape(N_CLS * CTX_PIECES, P, CTX_DIM)
  sfx_r = token_suffix.reshape(N_CLS * SFX_PIECES, P, CTX_DIM)
  tok_r = jnp.pad(tokenized_prompts, ((0, 0), (0, TOK_PAD - SEQ_LEN)))

  mesh = plsc.VectorSubcoreMesh(core_axis_name="c", subcore_axis_name="s")
  run = pl.kernel(
      _sc_gather_body,
      out_type=(
          jax.ShapeDtypeStruct((BATCH, SEQ_LEN, CTX_DIM), jnp.float32),
          jax.ShapeDtypeStruct((BATCH, TOK_PAD), jnp.int32),
      ),
      mesh=mesh,
      scratch_types=[
          pltpu.VMEM((BPW,), jnp.int32),
          pltpu.VMEM((NBUF, L), jnp.int32),
          pltpu.VMEM((L, 1, CTX_DIM), jnp.float32),
          pltpu.VMEM((L, TOK_PAD), jnp.int32),
          pltpu.VMEM((L, P, CTX_DIM), jnp.float32),
          pltpu.VMEM((L, P, CTX_DIM), jnp.float32),
          pltpu.VMEM((L, P, CTX_DIM), jnp.float32),
          [pltpu.SemaphoreType.DMA] * NBUF,
          [pltpu.SemaphoreType.DMA] * NBUF,
          pltpu.SemaphoreType.DMA,
          pltpu.SemaphoreType.DMA,
          pltpu.SemaphoreType.DMA,
          pltpu.SemaphoreType.DMA,
      ],
      compiler_params=pltpu.CompilerParams(use_tc_tiling_on_sc=False),
  )
  prompts, tok_padded = run(label, token_prefix, ctx_r, sfx_r, tok_r)
  return prompts, tok_padded[:, :SEQ_LEN]


def kernel(label, ctx, token_prefix, token_suffix, tokenized_prompts):
  return _prompt_gather(label, ctx, token_prefix, token_suffix,
                        tokenized_prompts)


# speedup vs baseline: 1.5426x; 1.4795x over previous
import jax
import jax.numpy as jnp
from jax import lax
from jax.experimental import pallas as pl
from jax.experimental.pallas import tpu as pltpu
from jax.experimental.pallas import tpu_sc as plsc

N_CLS = 100
N_CTX = 16
CTX_DIM = 512
SEQ_LEN = 77
SUFFIX_LEN = SEQ_LEN - 1 - N_CTX
BATCH = 1024

NC, NS, L = 2, 16, 16
NW = NC * NS
BPW = BATCH // NW

TOK_PAD = 128
ROW_STRIDE = 80
NBUF = 3


def _sc_gather_body(label_hbm, tbl_hbm, tok_hbm, out_hbm, tokout_hbm,
                    idx_v, ridx_v, stage0, stage1, stage2, tokbuf,
                    gsems, ssems, tg_sem, ts_sem):
  wid = lax.axis_index("s") * NC + lax.axis_index("c")
  base = wid * BPW
  stages = (stage0, stage1, stage2)

  pltpu.sync_copy(label_hbm.at[pl.ds(base, BPW)], idx_v)

  iota = lax.iota(jnp.int32, L)

  for g in range(BPW // L):
    lab = idx_v[pl.ds(g * L, L)]
    pos = iota * ROW_STRIDE + g * L * ROW_STRIDE
    for k in range(ROW_STRIDE):
      plsc.store_scatter(ridx_v, [pos + k],
                         lab * SEQ_LEN + min(k, SEQ_LEN - 1))

  tokc = None
  for g in range(BPW // L):
    b0 = base + g * L
    if g > 0:
      tokc.wait()
    tg = pltpu.make_async_copy(tok_hbm.at[idx_v.at[pl.ds(g * L, L)]],
                               tokbuf, tg_sem)
    tg.start()
    tg.wait()
    tokc = pltpu.make_async_copy(tokbuf, tokout_hbm.at[pl.ds(b0, L)], ts_sem)
    tokc.start()

  scatters = {}
  gathers = {}

  def start_gather(i):
    r = i % NBUF
    h0 = pltpu.make_async_copy(
        tbl_hbm.at[ridx_v.at[pl.ds(i * ROW_STRIDE, 72)]],
        stages[r].at[pl.ds(0, 72)], gsems[r])
    h1 = pltpu.make_async_copy(
        tbl_hbm.at[ridx_v.at[pl.ds(i * ROW_STRIDE + 72, 8)]],
        stages[r].at[pl.ds(72, 8)], gsems[r])
    h0.start()
    h1.start()
    gathers[i] = (h0, h1)

  start_gather(0)
  start_gather(1)
  for i in range(BPW):
    r = i % NBUF
    if i + 2 < BPW:
      if i + 2 >= NBUF:
        for sh in scatters.pop(i + 2 - NBUF):
          sh.wait()
      start_gather(i + 2)
    for h in gathers.pop(i):
      h.wait()
    sh0 = pltpu.make_async_copy(stages[r].at[pl.ds(0, 72)],
                                out_hbm.at[base + i, pl.ds(0, 72)], ssems[r])
    sh1 = pltpu.make_async_copy(stages[r].at[pl.ds(72, 5)],
                                out_hbm.at[base + i, pl.ds(72, 5)], ssems[r])
    sh0.start()
    sh1.start()
    scatters[i] = (sh0, sh1)

  for i in sorted(scatters):
    for sh in scatters[i]:
      sh.wait()
  tokc.wait()


@jax.jit
def _prompt_gather(label, ctx, token_prefix, token_suffix, tokenized_prompts):
  tbl = jnp.concatenate([token_prefix, ctx, token_suffix],
                        axis=1).reshape(N_CLS * SEQ_LEN, CTX_DIM)
  tok_r = jnp.pad(tokenized_prompts, ((0, 0), (0, TOK_PAD - SEQ_LEN)))

  mesh = plsc.VectorSubcoreMesh(core_axis_name="c", subcore_axis_name="s")
  run = pl.kernel(
      _sc_gather_body,
      out_type=(
          jax.ShapeDtypeStruct((BATCH, SEQ_LEN, CTX_DIM), jnp.float32),
          jax.ShapeDtypeStruct((BATCH, TOK_PAD), jnp.int32),
      ),
      mesh=mesh,
      scratch_types=[
          pltpu.VMEM((BPW,), jnp.int32),
          pltpu.VMEM((BPW * ROW_STRIDE,), jnp.int32),
          pltpu.VMEM((ROW_STRIDE, CTX_DIM), jnp.float32),
          pltpu.VMEM((ROW_STRIDE, CTX_DIM), jnp.float32),
          pltpu.VMEM((ROW_STRIDE, CTX_DIM), jnp.float32),
          pltpu.VMEM((L, TOK_PAD), jnp.int32),
          [pltpu.SemaphoreType.DMA] * NBUF,
          [pltpu.SemaphoreType.DMA] * NBUF,
          pltpu.SemaphoreType.DMA,
          pltpu.SemaphoreType.DMA,
      ],
      compiler_params=pltpu.CompilerParams(needs_layout_passes=False),
  )
  prompts, tok_padded = run(label, tbl, tok_r)
  return prompts, tok_padded[:, :SEQ_LEN]


def kernel(label, ctx, token_prefix, token_suffix, tokenized_prompts):
  return _prompt_gather(label, ctx, token_prefix, token_suffix,
                        tokenized_prompts)
